# baseline (speedup 1.0000x reference)
import functools

import jax
import jax.numpy as jnp
from jax import lax
from jax.experimental import pallas as pl
from jax.experimental.pallas import tpu as pltpu
from jax.experimental.pallas import tpu_sc as plsc

B, N, M = 4, 64, 100000
QUERY_DIM = 256
BUF0 = 64
CTX_DIM = 128
HEADS, DIM_HEAD = 8, 64
INNER = HEADS * DIM_HEAD
SCALE = DIM_HEAD ** (-0.5)

NW = 32
WPB = NW // B

TC_BLK = 2048
TC_NBLK = 12
TC_ROWS = TC_BLK * TC_NBLK
SC_ROWS = M - TC_ROWS

RPW = SC_ROWS // WPB
CHUNK = 512
NFULL = RPW // CHUNK
TAIL = RPW - NFULL * CHUNK
TAIL_G = (TAIL + 15) // 16
DUN = 8


def _proj_body(x0_ref, wq_ref, wqe_ref, e_ref):
    q0 = jnp.dot(x0_ref[...], wq_ref[...], preferred_element_type=jnp.float32)
    e_ref[...] = jnp.dot(q0, wqe_ref[...], preferred_element_type=jnp.float32)


_proj = pl.pallas_call(
    _proj_body,
    out_shape=jax.ShapeDtypeStruct((B, BUF0), jnp.float32),
)


def _upd(st, x, ix):
    m1, i1, m2, i2 = st
    lt1 = x < m1
    lt2 = x < m2
    m2n = jnp.where(lt1, m1, jnp.where(lt2, x, m2))
    i2n = jnp.where(lt1, i1, jnp.where(lt2, ix, i2))
    return (jnp.where(lt1, x, m1), jnp.where(lt1, ix, i1), m2n, i2n)


_sc_mesh = plsc.VectorSubcoreMesh(core_axis_name="c", subcore_axis_name="s")


@functools.partial(
    pl.kernel,
    out_type=(
        jax.ShapeDtypeStruct((NW, 128), jnp.float32),
        jax.ShapeDtypeStruct((NW, 128), jnp.int32),
    ),
    mesh=_sc_mesh,
    compiler_params=pltpu.CompilerParams(use_tc_tiling_on_sc=False,
                                         needs_layout_passes=False),
    scratch_types=[
        pltpu.VMEM((CHUNK, BUF0), jnp.float32),
        pltpu.VMEM((CHUNK, BUF0), jnp.float32),
        pltpu.VMEM((TAIL, BUF0), jnp.float32),
        pltpu.VMEM((BUF0,), jnp.float32),
        pltpu.VMEM((128,), jnp.float32),
        pltpu.VMEM((128,), jnp.int32),
        pltpu.SemaphoreType.DMA,
        pltpu.SemaphoreType.DMA,
        pltpu.SemaphoreType.DMA,
    ],
)
def _scan_topk(ctx_hbm, e_hbm, vals_hbm, idx_hbm,
               buf0, buf1, buft, e_v, val_v, idx_v, sem0, sem1, semt):
    wid = lax.axis_index("s") * 2 + lax.axis_index("c")
    b = wid // WPB
    row0 = TC_ROWS + (wid % WPB) * RPW

    pltpu.sync_copy(e_hbm.at[b], e_v)

    pltpu.async_copy(ctx_hbm.at[b, pl.ds(row0, CHUNK), pl.ds(0, BUF0)],
                     buf0, sem0)
    pltpu.async_copy(ctx_hbm.at[b, pl.ds(row0 + CHUNK, CHUNK), pl.ds(0, BUF0)],
                     buf1, sem1)
    pltpu.async_copy(
        ctx_hbm.at[b, pl.ds(row0 + NFULL * CHUNK, TAIL), pl.ds(0, BUF0)],
        buft, semt)

    iota = jnp.arange(16, dtype=jnp.int32)
    inf16 = jnp.full((16,), jnp.inf, jnp.float32)
    zi16 = jnp.zeros((16,), jnp.int32)
    z16 = jnp.zeros((16,), jnp.float32)
    state0 = tuple((inf16, zi16, inf16, zi16) for _ in range(4))

    def compute_chunk(buf, chunk_row0, state):
        def blk_body(blk, st):
            rowvecs = [iota + (blk * 64 + g * 16) for g in range(4)]

            def d_body(dblk, accs):
                accs = list(accs)
                for k in range(DUN):
                    dval = dblk * DUN + k
                    rot = (iota + dval) & 63
                    ev = plsc.load_gather(e_v, [rot])
                    for g in range(4):
                        xg = plsc.load_gather(buf, [rowvecs[g], rot])
                        df = xg - ev
                        accs[g] = accs[g] + df * df
                return tuple(accs)

            accs = lax.fori_loop(0, BUF0 // DUN, d_body, (z16, z16, z16, z16))
            return tuple(
                _upd(st[g], accs[g], rowvecs[g] + chunk_row0)
                for g in range(4))

        return lax.fori_loop(0, CHUNK // 64, blk_body, state)

    wait_src0 = ctx_hbm.at[0, pl.ds(0, CHUNK), pl.ds(0, BUF0)]

    def pair_body(j, state):
        c0 = 2 * j
        pltpu.make_async_copy(wait_src0, buf0, sem0).wait()
        state = compute_chunk(buf0, row0 + c0 * CHUNK, state)

        @pl.when(j < NFULL // 2 - 1)
        def _():
            pltpu.async_copy(
                ctx_hbm.at[b, pl.ds(row0 + (c0 + 2) * CHUNK, CHUNK),
                           pl.ds(0, BUF0)],
                buf0, sem0)

        pltpu.make_async_copy(wait_src0, buf1, sem1).wait()
        state = compute_chunk(buf1, row0 + (c0 + 1) * CHUNK, state)

        @pl.when(j < NFULL // 2 - 1)
        def _():
            pltpu.async_copy(
                ctx_hbm.at[b, pl.ds(row0 + (c0 + 3) * CHUNK, CHUNK),
                           pl.ds(0, BUF0)],
                buf1, sem1)

        return state

    state = lax.fori_loop(0, NFULL // 2, pair_body, state0)

    pltpu.make_async_copy(
        ctx_hbm.at[0, pl.ds(0, TAIL), pl.ds(0, BUF0)], buft, semt).wait()

    def tail_body(g, st0):
        rows = jnp.minimum(iota + g * 16, TAIL - 1)

        def d_body(dblk, acc):
            for k in range(DUN):
                dval = dblk * DUN + k
                rot = (iota + dval) & 63
                ev = plsc.load_gather(e_v, [rot])
                xg = plsc.load_gather(buft, [rows, rot])
                df = xg - ev
                acc = acc + df * df
            return acc

        acc = lax.fori_loop(0, BUF0 // DUN, d_body, z16)
        nvalid = TAIL - g * 16
        x = jnp.where(iota < nvalid, acc, jnp.inf)
        ix = row0 + NFULL * CHUNK + g * 16 + iota
        return _upd(st0, x, ix)

    st0 = lax.fori_loop(0, TAIL_G, tail_body, state[0])
    state = (st0,) + state[1:]

    for g in range(4):
        val_v[pl.ds(g * 16, 16)] = state[g][0]
        val_v[pl.ds(64 + g * 16, 16)] = state[g][2]
        idx_v[pl.ds(g * 16, 16)] = state[g][1]
        idx_v[pl.ds(64 + g * 16, 16)] = state[g][3]
    pltpu.sync_copy(val_v, vals_hbm.at[wid])
    pltpu.sync_copy(idx_v, idx_hbm.at[wid])


def _tc_scan_body(ctx_ref, e_ref, vals_ref, idx_ref, bufs, sems):
    bb = pl.program_id(0)
    blk = pl.program_id(1)
    step = bb * TC_NBLK + blk
    k = lax.rem(step, 2)

    def issue(bi, ii_, slot):
        pltpu.make_async_copy(
            ctx_ref.at[bi, pl.ds(ii_ * (TC_BLK // 8), TC_BLK // 8), :, 0, :],
            bufs.at[slot], sems.at[slot]).start()

    @pl.when(step == 0)
    def _():
        issue(0, 0, 0)

    nxt = step + 1

    @pl.when(nxt < B * TC_NBLK)
    def _():
        issue(nxt // TC_NBLK, lax.rem(nxt, TC_NBLK), lax.rem(nxt, 2))

    pltpu.make_async_copy(
        ctx_ref.at[0, pl.ds(0, TC_BLK // 8), :, 0, :],
        bufs.at[k], sems.at[k]).wait()
    e_row = e_ref[pl.ds(bb, 1), :]
    C8 = bufs[k].reshape(TC_BLK // 8, 8 * BUF0)
    e512 = jnp.concatenate([e_row] * 8, axis=1)
    T = C8 * (C8 - 2.0 * e512)
    E8 = (lax.broadcasted_iota(jnp.int32, (8 * BUF0, 8), 0) // BUF0
          == lax.broadcasted_iota(jnp.int32, (8 * BUF0, 8), 1)).astype(
              jnp.float32)
    ee = jnp.sum(e_row * e_row)
    s = jnp.dot(T, E8, preferred_element_type=jnp.float32) + ee
    ii = (lax.broadcasted_iota(jnp.int32, s.shape, 0) * 8
          + lax.broadcasted_iota(jnp.int32, s.shape, 1)
          + blk * TC_BLK).astype(jnp.float32)
    BIG = jnp.float32(3.0e38)
    m1 = jnp.min(s)
    i1 = jnp.min(jnp.where(s == m1, ii, BIG))
    s2 = jnp.where(ii == i1, BIG, s)
    m2 = jnp.min(s2)
    i2 = jnp.min(jnp.where(s2 == m2, ii, BIG))
    vals_ref[...] = jnp.stack([m1, m2]).reshape(1, 1, 2)
    idx_ref[...] = jnp.stack([i1, i2]).astype(jnp.int32).reshape(1, 1, 2)


_tc_scan = pl.pallas_call(
    _tc_scan_body,
    grid=(B, TC_NBLK),
    in_specs=[
        pl.BlockSpec(memory_space=pltpu.MemorySpace.HBM),
        pl.BlockSpec((B, BUF0), lambda b, i: (0, 0)),
    ],
    out_specs=(
        pl.BlockSpec((1, 1, 2), lambda b, i: (b * TC_NBLK + i, 0, 0)),
        pl.BlockSpec((1, 1, 2), lambda b, i: (b * TC_NBLK + i, 0, 0)),
    ),
    out_shape=(
        jax.ShapeDtypeStruct((B * TC_NBLK, 1, 2), jnp.float32),
        jax.ShapeDtypeStruct((B * TC_NBLK, 1, 2), jnp.int32),
    ),
    scratch_shapes=[
        pltpu.VMEM((2, TC_BLK // 8, 8, BUF0), jnp.float32),
        pltpu.SemaphoreType.DMA((2,)),
    ],
)


def _attn_body(x_ref, wq_ref, vals_ref, idx_ref, tcv_ref, tci_ref, ctx_ref,
               wk_ref, wv_ref, wo_ref, bo_ref, o_ref, rows_s, sem):
    f32 = jnp.float32
    BIG = jnp.float32(3.0e38)
    vals = jnp.concatenate(
        [vals_ref[...].reshape(B, WPB * 128), tcv_ref[...]], axis=1)
    idxf = jnp.concatenate(
        [idx_ref[...].reshape(B, WPB * 128), tci_ref[...]], axis=1).astype(f32)

    m1 = jnp.min(vals, axis=1, keepdims=True)
    i1 = jnp.min(jnp.where(vals == m1, idxf, BIG), axis=1, keepdims=True)
    vals2 = jnp.where(idxf == i1, BIG, vals)
    m2 = jnp.min(vals2, axis=1, keepdims=True)
    i2 = jnp.min(jnp.where(vals2 == m2, idxf, BIG), axis=1, keepdims=True)
    idx2 = jnp.concatenate([i1, i2], axis=1).astype(jnp.int32)

    for bb in range(B):
        for j in range(2):
            s = idx2[bb, j]
            pltpu.make_async_copy(
                ctx_ref.at[bb, pl.ds(s, 1), :],
                rows_s.at[bb, pl.ds(j, 1), :], sem).start()
    for _ in range(B * 2):
        pltpu.make_async_copy(
            ctx_ref.at[0, pl.ds(0, 1), :],
            rows_s.at[0, pl.ds(0, 1), :], sem).wait()

    rows = rows_s[...]
    creps = rows[:, :, :BUF0].reshape(B * 2, BUF0)
    clabels = rows[:, :, BUF0:].reshape(B * 2, BUF0)
    k = jnp.dot(clabels, wk_ref[...],
                preferred_element_type=f32).reshape(B, 2, INNER)
    v = jnp.dot(creps, wv_ref[...],
                preferred_element_type=f32).reshape(B, 2, INNER)
    q3 = jnp.dot(x_ref[...], wq_ref[...],
                 preferred_element_type=f32).reshape(B, N, INNER)

    E = (lax.broadcasted_iota(jnp.int32, (INNER, HEADS), 0) // DIM_HEAD
         == lax.broadcasted_iota(jnp.int32, (INNER, HEADS), 1)).astype(f32)

    sims = []
    for j in range(2):
        prod = (q3 * k[:, j][:, None, :]).reshape(B * N, INNER)
        sims.append(jnp.dot(prod, E, preferred_element_type=f32) * SCALE)
    mx = jnp.maximum(sims[0], sims[1])
    p0 = jnp.exp(sims[0] - mx)
    p1 = jnp.exp(sims[1] - mx)
    den = p0 + p1
    a0 = jnp.dot(p0 / den, E.T, preferred_element_type=f32).reshape(B, N, INNER)
    a1 = jnp.dot(p1 / den, E.T, preferred_element_type=f32).reshape(B, N, INNER)
    outi = a0 * v[:, 0][:, None, :] + a1 * v[:, 1][:, None, :]
    o_ref[...] = (jnp.dot(outi.reshape(B * N, INNER), wo_ref[...],
                          preferred_element_type=f32) + bo_ref[...])


_attn = pl.pallas_call(
    _attn_body,
    in_specs=[
        pl.BlockSpec(memory_space=pltpu.VMEM),
        pl.BlockSpec(memory_space=pltpu.VMEM),
        pl.BlockSpec(memory_space=pltpu.VMEM),
        pl.BlockSpec(memory_space=pltpu.VMEM),
        pl.BlockSpec(memory_space=pltpu.VMEM),
        pl.BlockSpec(memory_space=pltpu.VMEM),
        pl.BlockSpec(memory_space=pltpu.MemorySpace.HBM),
        pl.BlockSpec(memory_space=pltpu.VMEM),
        pl.BlockSpec(memory_space=pltpu.VMEM),
        pl.BlockSpec(memory_space=pltpu.VMEM),
        pl.BlockSpec(memory_space=pltpu.VMEM),
    ],
    out_shape=jax.ShapeDtypeStruct((B * N, QUERY_DIM), jnp.float32),
    scratch_shapes=[
        pltpu.VMEM((B, 2, CTX_DIM), jnp.float32),
        pltpu.SemaphoreType.DMA,
    ],
)


def kernel(x, context, W_q, W_k, W_v, W_qe, W_out, b_out, topk):
    del topk
    e = _proj(x[:, 0, :], W_q, W_qe)
    sc_vals, sc_idx = _scan_topk(context, e)
    tc_vals, tc_idx = _tc_scan(context.reshape(B, M // 8, 8, 2, BUF0), e)
    out = _attn(x.reshape(B * N, QUERY_DIM), W_q, sc_vals, sc_idx,
                tc_vals.reshape(B, 2 * TC_NBLK),
                tc_idx.reshape(B, 2 * TC_NBLK), context, W_k, W_v, W_out,
                b_out.reshape(1, QUERY_DIM))
    return out.reshape(B, N, QUERY_DIM)


# speedup vs baseline: 3.5827x; 3.5827x over previous
import functools

import jax
import jax.numpy as jnp
from jax import lax
from jax.experimental import pallas as pl
from jax.experimental.pallas import tpu as pltpu
from jax.experimental.pallas import tpu_sc as plsc

B, N, M = 4, 64, 100000
QUERY_DIM = 256
BUF0 = 64
CTX_DIM = 128
HEADS, DIM_HEAD = 8, 64
INNER = HEADS * DIM_HEAD
SCALE = DIM_HEAD ** (-0.5)

NW = 32
WPB = NW // B

TC_BLK = 2048
TC_NBLK = 8
TC_ROWS = TC_BLK * TC_NBLK
SC_ROWS = M - TC_ROWS

RPW = SC_ROWS // WPB
CHUNK = 512
NFULL = RPW // CHUNK
TAIL = RPW - NFULL * CHUNK
TAIL_G = (TAIL + 15) // 16
DUN = 8


def _proj_body(x0_ref, wq_ref, wqe_ref, e_ref):
    q0 = jnp.dot(x0_ref[...], wq_ref[...], preferred_element_type=jnp.float32)
    e_ref[...] = jnp.dot(q0, wqe_ref[...], preferred_element_type=jnp.float32)


_proj = pl.pallas_call(
    _proj_body,
    out_shape=jax.ShapeDtypeStruct((B, BUF0), jnp.float32),
)


def _upd(st, x, ix):
    m1, i1, m2, i2 = st
    lt1 = x < m1
    lt2 = x < m2
    m2n = jnp.where(lt1, m1, jnp.where(lt2, x, m2))
    i2n = jnp.where(lt1, i1, jnp.where(lt2, ix, i2))
    return (jnp.where(lt1, x, m1), jnp.where(lt1, ix, i1), m2n, i2n)


_sc_mesh = plsc.VectorSubcoreMesh(core_axis_name="c", subcore_axis_name="s")


@functools.partial(
    pl.kernel,
    out_type=(
        jax.ShapeDtypeStruct((NW, 128), jnp.float32),
        jax.ShapeDtypeStruct((NW, 128), jnp.int32),
    ),
    mesh=_sc_mesh,
    compiler_params=pltpu.CompilerParams(use_tc_tiling_on_sc=False,
                                         needs_layout_passes=False),
    scratch_types=[
        pltpu.VMEM((CHUNK, BUF0), jnp.float32),
        pltpu.VMEM((CHUNK, BUF0), jnp.float32),
        pltpu.VMEM((TAIL, BUF0), jnp.float32),
        pltpu.VMEM((BUF0,), jnp.float32),
        pltpu.VMEM((128,), jnp.float32),
        pltpu.VMEM((128,), jnp.int32),
        pltpu.SemaphoreType.DMA,
        pltpu.SemaphoreType.DMA,
        pltpu.SemaphoreType.DMA,
    ],
)
def _scan_topk(ctx_hbm, e_hbm, vals_hbm, idx_hbm,
               buf0, buf1, buft, e_v, val_v, idx_v, sem0, sem1, semt):
    wid = lax.axis_index("s") * 2 + lax.axis_index("c")
    b = wid // WPB
    row0 = TC_ROWS + (wid % WPB) * RPW

    pltpu.sync_copy(e_hbm.at[b], e_v)

    pltpu.async_copy(ctx_hbm.at[b, pl.ds(row0, CHUNK), pl.ds(0, BUF0)],
                     buf0, sem0)
    pltpu.async_copy(ctx_hbm.at[b, pl.ds(row0 + CHUNK, CHUNK), pl.ds(0, BUF0)],
                     buf1, sem1)
    pltpu.async_copy(
        ctx_hbm.at[b, pl.ds(row0 + NFULL * CHUNK, TAIL), pl.ds(0, BUF0)],
        buft, semt)

    iota = jnp.arange(16, dtype=jnp.int32)
    inf16 = jnp.full((16,), jnp.inf, jnp.float32)
    zi16 = jnp.zeros((16,), jnp.int32)
    z16 = jnp.zeros((16,), jnp.float32)
    state0 = tuple((inf16, zi16, inf16, zi16) for _ in range(4))

    def compute_chunk(buf, chunk_row0, state):
        def blk_body(blk, st):
            rowvecs = [iota + (blk * 64 + g * 16) for g in range(4)]

            def d_body(dblk, accs):
                accs = list(accs)
                for k in range(DUN):
                    dval = dblk * DUN + k
                    rot = (iota + dval) & 63
                    ev = plsc.load_gather(e_v, [rot])
                    for g in range(4):
                        xg = plsc.load_gather(buf, [rowvecs[g], rot])
                        df = xg - ev
                        accs[g] = accs[g] + df * df
                return tuple(accs)

            accs = lax.fori_loop(0, BUF0 // DUN, d_body, (z16, z16, z16, z16))
            return tuple(
                _upd(st[g], accs[g], rowvecs[g] + chunk_row0)
                for g in range(4))

        return lax.fori_loop(0, CHUNK // 64, blk_body, state)

    wait_src0 = ctx_hbm.at[0, pl.ds(0, CHUNK), pl.ds(0, BUF0)]

    def pair_body(j, state):
        c0 = 2 * j
        pltpu.make_async_copy(wait_src0, buf0, sem0).wait()
        state = compute_chunk(buf0, row0 + c0 * CHUNK, state)

        @pl.when(j < NFULL // 2 - 1)
        def _():
            pltpu.async_copy(
                ctx_hbm.at[b, pl.ds(row0 + (c0 + 2) * CHUNK, CHUNK),
                           pl.ds(0, BUF0)],
                buf0, sem0)

        pltpu.make_async_copy(wait_src0, buf1, sem1).wait()
        state = compute_chunk(buf1, row0 + (c0 + 1) * CHUNK, state)

        @pl.when(j < NFULL // 2 - 1)
        def _():
            pltpu.async_copy(
                ctx_hbm.at[b, pl.ds(row0 + (c0 + 3) * CHUNK, CHUNK),
                           pl.ds(0, BUF0)],
                buf1, sem1)

        return state

    state = lax.fori_loop(0, NFULL // 2, pair_body, state0)

    pltpu.make_async_copy(
        ctx_hbm.at[0, pl.ds(0, TAIL), pl.ds(0, BUF0)], buft, semt).wait()

    def tail_body(g, st0):
        rows = jnp.minimum(iota + g * 16, TAIL - 1)

        def d_body(dblk, acc):
            for k in range(DUN):
                dval = dblk * DUN + k
                rot = (iota + dval) & 63
                ev = plsc.load_gather(e_v, [rot])
                xg = plsc.load_gather(buft, [rows, rot])
                df = xg - ev
                acc = acc + df * df
            return acc

        acc = lax.fori_loop(0, BUF0 // DUN, d_body, z16)
        nvalid = TAIL - g * 16
        x = jnp.where(iota < nvalid, acc, jnp.inf)
        ix = row0 + NFULL * CHUNK + g * 16 + iota
        return _upd(st0, x, ix)

    st0 = lax.fori_loop(0, TAIL_G, tail_body, state[0])
    state = (st0,) + state[1:]

    for g in range(4):
        val_v[pl.ds(g * 16, 16)] = state[g][0]
        val_v[pl.ds(64 + g * 16, 16)] = state[g][2]
        idx_v[pl.ds(g * 16, 16)] = state[g][1]
        idx_v[pl.ds(64 + g * 16, 16)] = state[g][3]
    pltpu.sync_copy(val_v, vals_hbm.at[wid])
    pltpu.sync_copy(idx_v, idx_hbm.at[wid])


def _tc_scan_body(ctx_ref, e_ref, vals_ref, idx_ref, bufs, sems):
    bb = pl.program_id(0)
    blk = pl.program_id(1)
    step = bb * TC_NBLK + blk
    k = lax.rem(step, 2)

    def issue(bi, ii_, slot):
        pltpu.make_async_copy(
            ctx_ref.at[bi, pl.ds(ii_ * (TC_BLK // 4), TC_BLK // 4), :],
            bufs.at[slot], sems.at[slot]).start()

    @pl.when(step == 0)
    def _():
        issue(0, 0, 0)

    nxt = step + 1

    @pl.when(nxt < B * TC_NBLK)
    def _():
        issue(nxt // TC_NBLK, lax.rem(nxt, TC_NBLK), lax.rem(nxt, 2))

    pltpu.make_async_copy(
        ctx_ref.at[0, pl.ds(0, TC_BLK // 4), :],
        bufs.at[k], sems.at[k]).wait()
    e_row = e_ref[pl.ds(bb, 1), :]
    C4 = bufs[k]
    zz = jnp.zeros((1, BUF0), jnp.float32)
    e512 = jnp.concatenate([e_row, zz, e_row, zz, e_row, zz, e_row, zz],
                           axis=1)
    T = C4 * (C4 - 2.0 * e512)
    d_iota = lax.broadcasted_iota(jnp.int32, (4 * CTX_DIM, 4), 0)
    j_iota = lax.broadcasted_iota(jnp.int32, (4 * CTX_DIM, 4), 1)
    E4 = ((d_iota // CTX_DIM == j_iota)
          & (d_iota % CTX_DIM < BUF0)).astype(jnp.float32)
    ee = jnp.sum(e_row * e_row)
    s = jnp.dot(T, E4, preferred_element_type=jnp.float32) + ee
    ii = (lax.broadcasted_iota(jnp.int32, s.shape, 0) * 4
          + lax.broadcasted_iota(jnp.int32, s.shape, 1)
          + blk * TC_BLK).astype(jnp.float32)
    BIG = jnp.float32(3.0e38)
    m1 = jnp.min(s)
    i1 = jnp.min(jnp.where(s == m1, ii, BIG))
    s2 = jnp.where(ii == i1, BIG, s)
    m2 = jnp.min(s2)
    i2 = jnp.min(jnp.where(s2 == m2, ii, BIG))
    vals_ref[...] = jnp.stack([m1, m2]).reshape(1, 1, 2)
    idx_ref[...] = jnp.stack([i1, i2]).astype(jnp.int32).reshape(1, 1, 2)


_tc_scan = pl.pallas_call(
    _tc_scan_body,
    grid=(B, TC_NBLK),
    in_specs=[
        pl.BlockSpec(memory_space=pltpu.MemorySpace.HBM),
        pl.BlockSpec((B, BUF0), lambda b, i: (0, 0)),
    ],
    out_specs=(
        pl.BlockSpec((1, 1, 2), lambda b, i: (b * TC_NBLK + i, 0, 0)),
        pl.BlockSpec((1, 1, 2), lambda b, i: (b * TC_NBLK + i, 0, 0)),
    ),
    out_shape=(
        jax.ShapeDtypeStruct((B * TC_NBLK, 1, 2), jnp.float32),
        jax.ShapeDtypeStruct((B * TC_NBLK, 1, 2), jnp.int32),
    ),
    scratch_shapes=[
        pltpu.VMEM((2, TC_BLK // 4, 4 * CTX_DIM), jnp.float32),
        pltpu.SemaphoreType.DMA((2,)),
    ],
)


def _attn_body(x_ref, wq_ref, vals_ref, idx_ref, tcv_ref, tci_ref, ctx_ref,
               wk_ref, wv_ref, wo_ref, bo_ref, o_ref, rows_s, sem):
    f32 = jnp.float32
    BIG = jnp.float32(3.0e38)
    vals = jnp.concatenate(
        [vals_ref[...].reshape(B, WPB * 128), tcv_ref[...]], axis=1)
    idxf = jnp.concatenate(
        [idx_ref[...].reshape(B, WPB * 128), tci_ref[...]], axis=1).astype(f32)

    m1 = jnp.min(vals, axis=1, keepdims=True)
    i1 = jnp.min(jnp.where(vals == m1, idxf, BIG), axis=1, keepdims=True)
    vals2 = jnp.where(idxf == i1, BIG, vals)
    m2 = jnp.min(vals2, axis=1, keepdims=True)
    i2 = jnp.min(jnp.where(vals2 == m2, idxf, BIG), axis=1, keepdims=True)
    idx2 = jnp.concatenate([i1, i2], axis=1).astype(jnp.int32)

    for bb in range(B):
        for j in range(2):
            s = idx2[bb, j]
            pltpu.make_async_copy(
                ctx_ref.at[bb, pl.ds(s, 1), :],
                rows_s.at[bb, pl.ds(j, 1), :], sem).start()
    for _ in range(B * 2):
        pltpu.make_async_copy(
            ctx_ref.at[0, pl.ds(0, 1), :],
            rows_s.at[0, pl.ds(0, 1), :], sem).wait()

    rows = rows_s[...]
    creps = rows[:, :, :BUF0].reshape(B * 2, BUF0)
    clabels = rows[:, :, BUF0:].reshape(B * 2, BUF0)
    k = jnp.dot(clabels, wk_ref[...],
                preferred_element_type=f32).reshape(B, 2, INNER)
    v = jnp.dot(creps, wv_ref[...],
                preferred_element_type=f32).reshape(B, 2, INNER)
    q3 = jnp.dot(x_ref[...], wq_ref[...],
                 preferred_element_type=f32).reshape(B, N, INNER)

    E = (lax.broadcasted_iota(jnp.int32, (INNER, HEADS), 0) // DIM_HEAD
         == lax.broadcasted_iota(jnp.int32, (INNER, HEADS), 1)).astype(f32)

    sims = []
    for j in range(2):
        prod = (q3 * k[:, j][:, None, :]).reshape(B * N, INNER)
        sims.append(jnp.dot(prod, E, preferred_element_type=f32) * SCALE)
    mx = jnp.maximum(sims[0], sims[1])
    p0 = jnp.exp(sims[0] - mx)
    p1 = jnp.exp(sims[1] - mx)
    den = p0 + p1
    a0 = jnp.dot(p0 / den, E.T, preferred_element_type=f32).reshape(B, N, INNER)
    a1 = jnp.dot(p1 / den, E.T, preferred_element_type=f32).reshape(B, N, INNER)
    outi = a0 * v[:, 0][:, None, :] + a1 * v[:, 1][:, None, :]
    o_ref[...] = (jnp.dot(outi.reshape(B * N, INNER), wo_ref[...],
                          preferred_element_type=f32) + bo_ref[...])


_attn = pl.pallas_call(
    _attn_body,
    in_specs=[
        pl.BlockSpec(memory_space=pltpu.VMEM),
        pl.BlockSpec(memory_space=pltpu.VMEM),
        pl.BlockSpec(memory_space=pltpu.VMEM),
        pl.BlockSpec(memory_space=pltpu.VMEM),
        pl.BlockSpec(memory_space=pltpu.VMEM),
        pl.BlockSpec(memory_space=pltpu.VMEM),
        pl.BlockSpec(memory_space=pltpu.MemorySpace.HBM),
        pl.BlockSpec(memory_space=pltpu.VMEM),
        pl.BlockSpec(memory_space=pltpu.VMEM),
        pl.BlockSpec(memory_space=pltpu.VMEM),
        pl.BlockSpec(memory_space=pltpu.VMEM),
    ],
    out_shape=jax.ShapeDtypeStruct((B * N, QUERY_DIM), jnp.float32),
    scratch_shapes=[
        pltpu.VMEM((B, 2, CTX_DIM), jnp.float32),
        pltpu.SemaphoreType.DMA,
    ],
)


def kernel(x, context, W_q, W_k, W_v, W_qe, W_out, b_out, topk):
    del topk
    e = _proj(x[:, 0, :], W_q, W_qe)
    sc_vals, sc_idx = _scan_topk(context, e)
    tc_vals, tc_idx = _tc_scan(context.reshape(B, M // 4, 4 * CTX_DIM), e)
    out = _attn(x.reshape(B * N, QUERY_DIM), W_q, sc_vals, sc_idx,
                tc_vals.reshape(B, 2 * TC_NBLK),
                tc_idx.reshape(B, 2 * TC_NBLK), context, W_k, W_v, W_out,
                b_out.reshape(1, QUERY_DIM))
    return out.reshape(B, N, QUERY_DIM)


# speedup vs baseline: 11.6264x; 3.2452x over previous
import functools

import jax
import jax.numpy as jnp
from jax import lax
from jax.experimental import pallas as pl
from jax.experimental.pallas import tpu as pltpu
from jax.experimental.pallas import tpu_sc as plsc

B, N, M = 4, 64, 100000
QUERY_DIM = 256
BUF0 = 64
CTX_DIM = 128
HEADS, DIM_HEAD = 8, 64
INNER = HEADS * DIM_HEAD
SCALE = DIM_HEAD ** (-0.5)

NW = 32
WPB = NW // B

TC_BLK = 2048
TC_NBLK = 8
TC_ROWS = TC_BLK * TC_NBLK
SC_ROWS = M - TC_ROWS

RPW = SC_ROWS // WPB
CHUNK = 512
NFULL = RPW // CHUNK
TAIL = RPW - NFULL * CHUNK
TAIL_G = (TAIL + 15) // 16
DUN = 8


def _proj_body(x0_ref, wq_ref, wqe_ref, e_ref):
    q0 = jnp.dot(x0_ref[...], wq_ref[...], preferred_element_type=jnp.float32)
    e_ref[...] = jnp.dot(q0, wqe_ref[...], preferred_element_type=jnp.float32)


_proj = pl.pallas_call(
    _proj_body,
    out_shape=jax.ShapeDtypeStruct((B, BUF0), jnp.float32),
)


def _upd(st, x, ix):
    m1, i1, m2, i2 = st
    lt1 = x < m1
    lt2 = x < m2
    m2n = jnp.where(lt1, m1, jnp.where(lt2, x, m2))
    i2n = jnp.where(lt1, i1, jnp.where(lt2, ix, i2))
    return (jnp.where(lt1, x, m1), jnp.where(lt1, ix, i1), m2n, i2n)


_sc_mesh = plsc.VectorSubcoreMesh(core_axis_name="c", subcore_axis_name="s")


@functools.partial(
    pl.kernel,
    out_type=(
        jax.ShapeDtypeStruct((NW, 128), jnp.float32),
        jax.ShapeDtypeStruct((NW, 128), jnp.int32),
    ),
    mesh=_sc_mesh,
    compiler_params=pltpu.CompilerParams(use_tc_tiling_on_sc=False,
                                         needs_layout_passes=False),
    scratch_types=[
        pltpu.VMEM((CHUNK, BUF0), jnp.float32),
        pltpu.VMEM((CHUNK, BUF0), jnp.float32),
        pltpu.VMEM((TAIL, BUF0), jnp.float32),
        pltpu.VMEM((BUF0,), jnp.float32),
        pltpu.VMEM((128,), jnp.float32),
        pltpu.VMEM((128,), jnp.int32),
        pltpu.SemaphoreType.DMA,
        pltpu.SemaphoreType.DMA,
        pltpu.SemaphoreType.DMA,
    ],
)
def _scan_topk(ctx_hbm, e_hbm, vals_hbm, idx_hbm,
               buf0, buf1, buft, e_v, val_v, idx_v, sem0, sem1, semt):
    wid = lax.axis_index("s") * 2 + lax.axis_index("c")
    b = wid // WPB
    row0 = TC_ROWS + (wid % WPB) * RPW

    pltpu.sync_copy(e_hbm.at[b], e_v)

    pltpu.async_copy(ctx_hbm.at[b, pl.ds(row0, CHUNK), pl.ds(0, BUF0)],
                     buf0, sem0)
    pltpu.async_copy(ctx_hbm.at[b, pl.ds(row0 + CHUNK, CHUNK), pl.ds(0, BUF0)],
                     buf1, sem1)
    pltpu.async_copy(
        ctx_hbm.at[b, pl.ds(row0 + NFULL * CHUNK, TAIL), pl.ds(0, BUF0)],
        buft, semt)

    iota = jnp.arange(16, dtype=jnp.int32)
    inf16 = jnp.full((16,), jnp.inf, jnp.float32)
    zi16 = jnp.zeros((16,), jnp.int32)
    z16 = jnp.zeros((16,), jnp.float32)
    state0 = tuple((inf16, zi16, inf16, zi16) for _ in range(4))

    def compute_chunk(buf, chunk_row0, state):
        def blk_body(blk, st):
            rowvecs = [iota + (blk * 64 + g * 16) for g in range(4)]

            def d_body(dblk, accs):
                accs = list(accs)
                for k in range(DUN):
                    dval = dblk * DUN + k
                    rot = (iota + dval) & 63
                    ev = plsc.load_gather(e_v, [rot])
                    for g in range(4):
                        xg = plsc.load_gather(buf, [rowvecs[g], rot])
                        df = xg - ev
                        accs[g] = accs[g] + df * df
                return tuple(accs)

            accs = lax.fori_loop(0, BUF0 // DUN, d_body, (z16, z16, z16, z16))
            return tuple(
                _upd(st[g], accs[g], rowvecs[g] + chunk_row0)
                for g in range(4))

        return lax.fori_loop(0, CHUNK // 64, blk_body, state)

    wait_src0 = ctx_hbm.at[0, pl.ds(0, CHUNK), pl.ds(0, BUF0)]

    def pair_body(j, state):
        c0 = 2 * j
        pltpu.make_async_copy(wait_src0, buf0, sem0).wait()
        state = compute_chunk(buf0, row0 + c0 * CHUNK, state)

        @pl.when(j < NFULL // 2 - 1)
        def _():
            pltpu.async_copy(
                ctx_hbm.at[b, pl.ds(row0 + (c0 + 2) * CHUNK, CHUNK),
                           pl.ds(0, BUF0)],
                buf0, sem0)

        pltpu.make_async_copy(wait_src0, buf1, sem1).wait()
        state = compute_chunk(buf1, row0 + (c0 + 1) * CHUNK, state)

        @pl.when(j < NFULL // 2 - 1)
        def _():
            pltpu.async_copy(
                ctx_hbm.at[b, pl.ds(row0 + (c0 + 3) * CHUNK, CHUNK),
                           pl.ds(0, BUF0)],
                buf1, sem1)

        return state

    state = lax.fori_loop(0, NFULL // 2, pair_body, state0)

    pltpu.make_async_copy(
        ctx_hbm.at[0, pl.ds(0, TAIL), pl.ds(0, BUF0)], buft, semt).wait()

    def tail_body(g, st0):
        rows = jnp.minimum(iota + g * 16, TAIL - 1)

        def d_body(dblk, acc):
            for k in range(DUN):
                dval = dblk * DUN + k
                rot = (iota + dval) & 63
                ev = plsc.load_gather(e_v, [rot])
                xg = plsc.load_gather(buft, [rows, rot])
                df = xg - ev
                acc = acc + df * df
            return acc

        acc = lax.fori_loop(0, BUF0 // DUN, d_body, z16)
        nvalid = TAIL - g * 16
        x = jnp.where(iota < nvalid, acc, jnp.inf)
        ix = row0 + NFULL * CHUNK + g * 16 + iota
        return _upd(st0, x, ix)

    st0 = lax.fori_loop(0, TAIL_G, tail_body, state[0])
    state = (st0,) + state[1:]

    for g in range(4):
        val_v[pl.ds(g * 16, 16)] = state[g][0]
        val_v[pl.ds(64 + g * 16, 16)] = state[g][2]
        idx_v[pl.ds(g * 16, 16)] = state[g][1]
        idx_v[pl.ds(64 + g * 16, 16)] = state[g][3]
    pltpu.sync_copy(val_v, vals_hbm.at[wid])
    pltpu.sync_copy(idx_v, idx_hbm.at[wid])


def _tc_scan_body(ctx_ref, e_ref, vals_ref, idx_ref, bufs, sems):
    bb = pl.program_id(0)
    blk = pl.program_id(1)
    step = bb * TC_NBLK + blk
    k = lax.rem(step, 2)

    def issue(bi, ii_, slot):
        pltpu.make_async_copy(
            ctx_ref.at[bi, pl.ds(ii_ * TC_BLK, TC_BLK), :],
            bufs.at[slot], sems.at[slot]).start()

    @pl.when(step == 0)
    def _():
        issue(0, 0, 0)

    nxt = step + 1

    @pl.when(nxt < B * TC_NBLK)
    def _():
        issue(nxt // TC_NBLK, lax.rem(nxt, TC_NBLK), lax.rem(nxt, 2))

    pltpu.make_async_copy(
        ctx_ref.at[0, pl.ds(0, TC_BLK), :],
        bufs.at[k], sems.at[k]).wait()
    e_row = e_ref[pl.ds(bb, 1), :]
    C = bufs[k]
    zz = jnp.zeros((1, BUF0), jnp.float32)
    e128 = jnp.concatenate([e_row, zz], axis=1)
    T = C * (C - 2.0 * e128)
    E_sel = (lax.broadcasted_iota(jnp.int32, (CTX_DIM, CTX_DIM), 0)
             < BUF0).astype(jnp.float32)
    ee = jnp.sum(e_row * e_row)
    s = jnp.dot(T, E_sel, preferred_element_type=jnp.float32) + ee
    ii = (lax.broadcasted_iota(jnp.int32, s.shape, 0)
          + blk * TC_BLK).astype(jnp.float32)
    BIG = jnp.float32(3.0e38)
    m1 = jnp.min(s)
    i1 = jnp.min(jnp.where(s == m1, ii, BIG))
    s2 = jnp.where(ii == i1, BIG, s)
    m2 = jnp.min(s2)
    i2 = jnp.min(jnp.where(s2 == m2, ii, BIG))
    vals_ref[...] = jnp.stack([m1, m2]).reshape(1, 1, 2)
    idx_ref[...] = jnp.stack([i1, i2]).astype(jnp.int32).reshape(1, 1, 2)


_tc_scan = pl.pallas_call(
    _tc_scan_body,
    grid=(B, TC_NBLK),
    in_specs=[
        pl.BlockSpec(memory_space=pltpu.MemorySpace.HBM),
        pl.BlockSpec((B, BUF0), lambda b, i: (0, 0)),
    ],
    out_specs=(
        pl.BlockSpec((1, 1, 2), lambda b, i: (b * TC_NBLK + i, 0, 0)),
        pl.BlockSpec((1, 1, 2), lambda b, i: (b * TC_NBLK + i, 0, 0)),
    ),
    out_shape=(
        jax.ShapeDtypeStruct((B * TC_NBLK, 1, 2), jnp.float32),
        jax.ShapeDtypeStruct((B * TC_NBLK, 1, 2), jnp.int32),
    ),
    scratch_shapes=[
        pltpu.VMEM((2, TC_BLK, CTX_DIM), jnp.float32),
        pltpu.SemaphoreType.DMA((2,)),
    ],
)


def _attn_body(x_ref, wq_ref, vals_ref, idx_ref, tcv_ref, tci_ref, ctx_ref,
               wk_ref, wv_ref, wo_ref, bo_ref, o_ref, rows_s, sem):
    f32 = jnp.float32
    BIG = jnp.float32(3.0e38)
    vals = jnp.concatenate(
        [vals_ref[...].reshape(B, WPB * 128), tcv_ref[...]], axis=1)
    idxf = jnp.concatenate(
        [idx_ref[...].reshape(B, WPB * 128), tci_ref[...]], axis=1).astype(f32)

    m1 = jnp.min(vals, axis=1, keepdims=True)
    i1 = jnp.min(jnp.where(vals == m1, idxf, BIG), axis=1, keepdims=True)
    vals2 = jnp.where(idxf == i1, BIG, vals)
    m2 = jnp.min(vals2, axis=1, keepdims=True)
    i2 = jnp.min(jnp.where(vals2 == m2, idxf, BIG), axis=1, keepdims=True)
    idx2 = jnp.concatenate([i1, i2], axis=1).astype(jnp.int32)

    for bb in range(B):
        for j in range(2):
            s = idx2[bb, j]
            pltpu.make_async_copy(
                ctx_ref.at[bb, pl.ds(s, 1), :],
                rows_s.at[bb, pl.ds(j, 1), :], sem).start()
    for _ in range(B * 2):
        pltpu.make_async_copy(
            ctx_ref.at[0, pl.ds(0, 1), :],
            rows_s.at[0, pl.ds(0, 1), :], sem).wait()

    rows = rows_s[...]
    creps = rows[:, :, :BUF0].reshape(B * 2, BUF0)
    clabels = rows[:, :, BUF0:].reshape(B * 2, BUF0)
    k = jnp.dot(clabels, wk_ref[...],
                preferred_element_type=f32).reshape(B, 2, INNER)
    v = jnp.dot(creps, wv_ref[...],
                preferred_element_type=f32).reshape(B, 2, INNER)
    q3 = jnp.dot(x_ref[...], wq_ref[...],
                 preferred_element_type=f32).reshape(B, N, INNER)

    E = (lax.broadcasted_iota(jnp.int32, (INNER, HEADS), 0) // DIM_HEAD
         == lax.broadcasted_iota(jnp.int32, (INNER, HEADS), 1)).astype(f32)

    sims = []
    for j in range(2):
        prod = (q3 * k[:, j][:, None, :]).reshape(B * N, INNER)
        sims.append(jnp.dot(prod, E, preferred_element_type=f32) * SCALE)
    mx = jnp.maximum(sims[0], sims[1])
    p0 = jnp.exp(sims[0] - mx)
    p1 = jnp.exp(sims[1] - mx)
    den = p0 + p1
    a0 = jnp.dot(p0 / den, E.T, preferred_element_type=f32).reshape(B, N, INNER)
    a1 = jnp.dot(p1 / den, E.T, preferred_element_type=f32).reshape(B, N, INNER)
    outi = a0 * v[:, 0][:, None, :] + a1 * v[:, 1][:, None, :]
    o_ref[...] = (jnp.dot(outi.reshape(B * N, INNER), wo_ref[...],
                          preferred_element_type=f32) + bo_ref[...])


_attn = pl.pallas_call(
    _attn_body,
    in_specs=[
        pl.BlockSpec(memory_space=pltpu.VMEM),
        pl.BlockSpec(memory_space=pltpu.VMEM),
        pl.BlockSpec(memory_space=pltpu.VMEM),
        pl.BlockSpec(memory_space=pltpu.VMEM),
        pl.BlockSpec(memory_space=pltpu.VMEM),
        pl.BlockSpec(memory_space=pltpu.VMEM),
        pl.BlockSpec(memory_space=pltpu.MemorySpace.HBM),
        pl.BlockSpec(memory_space=pltpu.VMEM),
        pl.BlockSpec(memory_space=pltpu.VMEM),
        pl.BlockSpec(memory_space=pltpu.VMEM),
        pl.BlockSpec(memory_space=pltpu.VMEM),
    ],
    out_shape=jax.ShapeDtypeStruct((B * N, QUERY_DIM), jnp.float32),
    scratch_shapes=[
        pltpu.VMEM((B, 2, CTX_DIM), jnp.float32),
        pltpu.SemaphoreType.DMA,
    ],
)


def kernel(x, context, W_q, W_k, W_v, W_qe, W_out, b_out, topk):
    del topk
    e = _proj(x[:, 0, :], W_q, W_qe)
    sc_vals, sc_idx = _scan_topk(context, e)
    tc_vals, tc_idx = _tc_scan(context, e)
    out = _attn(x.reshape(B * N, QUERY_DIM), W_q, sc_vals, sc_idx,
                tc_vals.reshape(B, 2 * TC_NBLK),
                tc_idx.reshape(B, 2 * TC_NBLK), context, W_k, W_v, W_out,
                b_out.reshape(1, QUERY_DIM))
    return out.reshape(B, N, QUERY_DIM)


# speedup vs baseline: 12.2260x; 1.0516x over previous
import functools

import jax
import jax.numpy as jnp
from jax import lax
from jax.experimental import pallas as pl
from jax.experimental.pallas import tpu as pltpu
from jax.experimental.pallas import tpu_sc as plsc

B, N, M = 4, 64, 100000
QUERY_DIM = 256
BUF0 = 64
CTX_DIM = 128
HEADS, DIM_HEAD = 8, 64
INNER = HEADS * DIM_HEAD
SCALE = DIM_HEAD ** (-0.5)

NW = 32
WPB = NW // B

TC_BLK = 2048
TC_NBLK = 0
TC_ROWS = TC_BLK * TC_NBLK
SC_ROWS = M - TC_ROWS

RPW = SC_ROWS // WPB
CHUNK = 512
NFULL = RPW // CHUNK
TAIL = RPW - NFULL * CHUNK
TAIL_G = (TAIL + 15) // 16
DUN = 8


def _proj_body(x0_ref, wq_ref, wqe_ref, e_ref):
    q0 = jnp.dot(x0_ref[...], wq_ref[...], preferred_element_type=jnp.float32)
    e_ref[...] = jnp.dot(q0, wqe_ref[...], preferred_element_type=jnp.float32)


_proj = pl.pallas_call(
    _proj_body,
    out_shape=jax.ShapeDtypeStruct((B, BUF0), jnp.float32),
)


def _upd(st, x, ix):
    m1, i1, m2, i2 = st
    lt1 = x < m1
    lt2 = x < m2
    m2n = jnp.where(lt1, m1, jnp.where(lt2, x, m2))
    i2n = jnp.where(lt1, i1, jnp.where(lt2, ix, i2))
    return (jnp.where(lt1, x, m1), jnp.where(lt1, ix, i1), m2n, i2n)


_sc_mesh = plsc.VectorSubcoreMesh(core_axis_name="c", subcore_axis_name="s")


@functools.partial(
    pl.kernel,
    out_type=(
        jax.ShapeDtypeStruct((NW, 128), jnp.float32),
        jax.ShapeDtypeStruct((NW, 128), jnp.int32),
    ),
    mesh=_sc_mesh,
    compiler_params=pltpu.CompilerParams(use_tc_tiling_on_sc=False,
                                         needs_layout_passes=False),
    scratch_types=[
        pltpu.VMEM((CHUNK, BUF0), jnp.float32),
        pltpu.VMEM((CHUNK, BUF0), jnp.float32),
        pltpu.VMEM((CHUNK, BUF0), jnp.float32),
        pltpu.VMEM((TAIL, BUF0), jnp.float32),
        pltpu.VMEM((BUF0,), jnp.float32),
        pltpu.VMEM((128,), jnp.float32),
        pltpu.VMEM((128,), jnp.int32),
        pltpu.SemaphoreType.DMA,
        pltpu.SemaphoreType.DMA,
        pltpu.SemaphoreType.DMA,
        pltpu.SemaphoreType.DMA,
    ],
)
def _scan_topk(ctx_hbm, e_hbm, vals_hbm, idx_hbm,
               buf0, buf1, buf2, buft, e_v, val_v, idx_v,
               sem0, sem1, sem2, semt):
    wid = lax.axis_index("s") * 2 + lax.axis_index("c")
    b = wid // WPB
    row0 = TC_ROWS + (wid % WPB) * RPW

    pltpu.sync_copy(e_hbm.at[b], e_v)

    pltpu.async_copy(ctx_hbm.at[b, pl.ds(row0, CHUNK), pl.ds(0, BUF0)],
                     buf0, sem0)
    pltpu.async_copy(ctx_hbm.at[b, pl.ds(row0 + CHUNK, CHUNK), pl.ds(0, BUF0)],
                     buf1, sem1)
    pltpu.async_copy(
        ctx_hbm.at[b, pl.ds(row0 + 2 * CHUNK, CHUNK), pl.ds(0, BUF0)],
        buf2, sem2)
    pltpu.async_copy(
        ctx_hbm.at[b, pl.ds(row0 + NFULL * CHUNK, TAIL), pl.ds(0, BUF0)],
        buft, semt)

    iota = jnp.arange(16, dtype=jnp.int32)
    inf16 = jnp.full((16,), jnp.inf, jnp.float32)
    zi16 = jnp.zeros((16,), jnp.int32)
    z16 = jnp.zeros((16,), jnp.float32)
    state0 = tuple((inf16, zi16, inf16, zi16) for _ in range(4))

    def compute_chunk(buf, chunk_row0, state):
        def blk_body(blk, st):
            rowvecs = [iota + (blk * 64 + g * 16) for g in range(4)]

            def d_body(dblk, accs):
                accs = list(accs)
                for k in range(DUN):
                    dval = dblk * DUN + k
                    rot = (iota + dval) & 63
                    ev = plsc.load_gather(e_v, [rot])
                    for g in range(4):
                        xg = plsc.load_gather(buf, [rowvecs[g], rot])
                        df = xg - ev
                        accs[g] = accs[g] + df * df
                return tuple(accs)

            accs = lax.fori_loop(0, BUF0 // DUN, d_body, (z16, z16, z16, z16))
            return tuple(
                _upd(st[g], accs[g], rowvecs[g] + chunk_row0)
                for g in range(4))

        return lax.fori_loop(0, CHUNK // 64, blk_body, state)

    wait_src0 = ctx_hbm.at[0, pl.ds(0, CHUNK), pl.ds(0, BUF0)]

    def triple_body(j, state):
        c0 = 3 * j
        for s, (buf, sem) in enumerate(
                ((buf0, sem0), (buf1, sem1), (buf2, sem2))):
            pltpu.make_async_copy(wait_src0, buf, sem).wait()
            state = compute_chunk(buf, row0 + (c0 + s) * CHUNK, state)

            @pl.when(j < NFULL // 3 - 1)
            def _(buf=buf, sem=sem, s=s):
                pltpu.async_copy(
                    ctx_hbm.at[b, pl.ds(row0 + (c0 + s + 3) * CHUNK, CHUNK),
                               pl.ds(0, BUF0)],
                    buf, sem)

        return state

    state = lax.fori_loop(0, NFULL // 3, triple_body, state0)

    pltpu.make_async_copy(
        ctx_hbm.at[0, pl.ds(0, TAIL), pl.ds(0, BUF0)], buft, semt).wait()

    def tail_body(g, st0):
        rows = jnp.minimum(iota + g * 16, TAIL - 1)

        def d_body(dblk, acc):
            for k in range(DUN):
                dval = dblk * DUN + k
                rot = (iota + dval) & 63
                ev = plsc.load_gather(e_v, [rot])
                xg = plsc.load_gather(buft, [rows, rot])
                df = xg - ev
                acc = acc + df * df
            return acc

        acc = lax.fori_loop(0, BUF0 // DUN, d_body, z16)
        nvalid = TAIL - g * 16
        x = jnp.where(iota < nvalid, acc, jnp.inf)
        ix = row0 + NFULL * CHUNK + g * 16 + iota
        return _upd(st0, x, ix)

    st0 = lax.fori_loop(0, TAIL_G, tail_body, state[0])
    state = (st0,) + state[1:]

    for g in range(4):
        val_v[pl.ds(g * 16, 16)] = state[g][0]
        val_v[pl.ds(64 + g * 16, 16)] = state[g][2]
        idx_v[pl.ds(g * 16, 16)] = state[g][1]
        idx_v[pl.ds(64 + g * 16, 16)] = state[g][3]
    pltpu.sync_copy(val_v, vals_hbm.at[wid])
    pltpu.sync_copy(idx_v, idx_hbm.at[wid])


def _unused_tc_scan_body(ctx_ref, e_ref, vals_ref, idx_ref, bufs, sems):
    bb = pl.program_id(0)
    blk = pl.program_id(1)
    step = bb * TC_NBLK + blk
    k = lax.rem(step, 2)

    def issue(bi, ii, slot):
        pltpu.make_async_copy(
            ctx_ref.at[bi, pl.ds(ii * TC_BLK, TC_BLK), :],
            bufs.at[slot], sems.at[slot]).start()

    @pl.when(step == 0)
    def _():
        issue(0, 0, 0)

    nxt = step + 1

    @pl.when(nxt < B * TC_NBLK)
    def _():
        issue(nxt // TC_NBLK, lax.rem(nxt, TC_NBLK), lax.rem(nxt, 2))

    pltpu.make_async_copy(
        ctx_ref.at[0, pl.ds(0, TC_BLK), :],
        bufs.at[k], sems.at[k]).wait()
    C = bufs[k][:, :BUF0]
    e_row = e_ref[pl.ds(bb, 1), :]
    ones = jnp.ones((BUF0, 1), jnp.float32)
    sq = jnp.dot(C * C, ones, preferred_element_type=jnp.float32)
    ce = jnp.dot(C, e_row.reshape(BUF0, 1),
                 preferred_element_type=jnp.float32)
    ee = jnp.sum(e_row * e_row)
    s = (sq - 2.0 * ce + ee).reshape(TC_BLK // 128, 128)
    ii = (lax.broadcasted_iota(jnp.int32, s.shape, 0) * 128
          + lax.broadcasted_iota(jnp.int32, s.shape, 1)
          + blk * TC_BLK).astype(jnp.float32)
    BIG = jnp.float32(3.0e38)
    m1 = jnp.min(s)
    i1 = jnp.min(jnp.where(s == m1, ii, BIG))
    s2 = jnp.where(ii == i1, BIG, s)
    m2 = jnp.min(s2)
    i2 = jnp.min(jnp.where(s2 == m2, ii, BIG))
    vals_ref[...] = jnp.stack([m1, m2]).reshape(1, 1, 2)
    idx_ref[...] = jnp.stack([i1, i2]).astype(jnp.int32).reshape(1, 1, 2)


_tc_scan = None if TC_NBLK == 0 else pl.pallas_call(
    _unused_tc_scan_body,
    grid=(B, TC_NBLK),
    in_specs=[
        pl.BlockSpec(memory_space=pltpu.MemorySpace.HBM),
        pl.BlockSpec((B, BUF0), lambda b, i: (0, 0)),
    ],
    out_specs=(
        pl.BlockSpec((1, 1, 2), lambda b, i: (b * TC_NBLK + i, 0, 0)),
        pl.BlockSpec((1, 1, 2), lambda b, i: (b * TC_NBLK + i, 0, 0)),
    ),
    out_shape=(
        jax.ShapeDtypeStruct((B * TC_NBLK, 1, 2), jnp.float32),
        jax.ShapeDtypeStruct((B * TC_NBLK, 1, 2), jnp.int32),
    ),
    scratch_shapes=[
        pltpu.VMEM((2, TC_BLK, CTX_DIM), jnp.float32),
        pltpu.SemaphoreType.DMA((2,)),
    ],
)


def _attn_body(x_ref, wq_ref, vals_ref, idx_ref, ctx_ref, wk_ref, wv_ref,
               wo_ref, bo_ref, o_ref, rows_s, sem):
    f32 = jnp.float32
    BIG = jnp.float32(3.0e38)
    vals = vals_ref[...].reshape(B, WPB * 128)
    idxf = idx_ref[...].reshape(B, WPB * 128).astype(f32)

    m1 = jnp.min(vals, axis=1, keepdims=True)
    i1 = jnp.min(jnp.where(vals == m1, idxf, BIG), axis=1, keepdims=True)
    vals2 = jnp.where(idxf == i1, BIG, vals)
    m2 = jnp.min(vals2, axis=1, keepdims=True)
    i2 = jnp.min(jnp.where(vals2 == m2, idxf, BIG), axis=1, keepdims=True)
    idx2 = jnp.concatenate([i1, i2], axis=1).astype(jnp.int32)

    for bb in range(B):
        for j in range(2):
            s = idx2[bb, j]
            pltpu.make_async_copy(
                ctx_ref.at[bb, pl.ds(s, 1), :],
                rows_s.at[bb, pl.ds(j, 1), :], sem).start()
    for _ in range(B * 2):
        pltpu.make_async_copy(
            ctx_ref.at[0, pl.ds(0, 1), :],
            rows_s.at[0, pl.ds(0, 1), :], sem).wait()

    rows = rows_s[...]
    creps = rows[:, :, :BUF0].reshape(B * 2, BUF0)
    clabels = rows[:, :, BUF0:].reshape(B * 2, BUF0)
    k = jnp.dot(clabels, wk_ref[...],
                preferred_element_type=f32).reshape(B, 2, INNER)
    v = jnp.dot(creps, wv_ref[...],
                preferred_element_type=f32).reshape(B, 2, INNER)
    q3 = jnp.dot(x_ref[...], wq_ref[...],
                 preferred_element_type=f32).reshape(B, N, INNER)

    E = (lax.broadcasted_iota(jnp.int32, (INNER, HEADS), 0) // DIM_HEAD
         == lax.broadcasted_iota(jnp.int32, (INNER, HEADS), 1)).astype(f32)

    sims = []
    for j in range(2):
        prod = (q3 * k[:, j][:, None, :]).reshape(B * N, INNER)
        sims.append(jnp.dot(prod, E, preferred_element_type=f32) * SCALE)
    mx = jnp.maximum(sims[0], sims[1])
    p0 = jnp.exp(sims[0] - mx)
    p1 = jnp.exp(sims[1] - mx)
    den = p0 + p1
    a0 = jnp.dot(p0 / den, E.T, preferred_element_type=f32).reshape(B, N, INNER)
    a1 = jnp.dot(p1 / den, E.T, preferred_element_type=f32).reshape(B, N, INNER)
    outi = a0 * v[:, 0][:, None, :] + a1 * v[:, 1][:, None, :]
    o_ref[...] = (jnp.dot(outi.reshape(B * N, INNER), wo_ref[...],
                          preferred_element_type=f32) + bo_ref[...])


_attn = pl.pallas_call(
    _attn_body,
    in_specs=[
        pl.BlockSpec(memory_space=pltpu.VMEM),
        pl.BlockSpec(memory_space=pltpu.VMEM),
        pl.BlockSpec(memory_space=pltpu.VMEM),
        pl.BlockSpec(memory_space=pltpu.VMEM),
        pl.BlockSpec(memory_space=pltpu.MemorySpace.HBM),
        pl.BlockSpec(memory_space=pltpu.VMEM),
        pl.BlockSpec(memory_space=pltpu.VMEM),
        pl.BlockSpec(memory_space=pltpu.VMEM),
        pl.BlockSpec(memory_space=pltpu.VMEM),
    ],
    out_shape=jax.ShapeDtypeStruct((B * N, QUERY_DIM), jnp.float32),
    scratch_shapes=[
        pltpu.VMEM((B, 2, CTX_DIM), jnp.float32),
        pltpu.SemaphoreType.DMA,
    ],
)


def kernel(x, context, W_q, W_k, W_v, W_qe, W_out, b_out, topk):
    del topk
    e = _proj(x[:, 0, :], W_q, W_qe)
    sc_vals, sc_idx = _scan_topk(context, e)
    out = _attn(x.reshape(B * N, QUERY_DIM), W_q, sc_vals, sc_idx,
                context, W_k, W_v, W_out, b_out.reshape(1, QUERY_DIM))
    return out.reshape(B, N, QUERY_DIM)


# speedup vs baseline: 12.6411x; 1.0340x over previous
import functools

import jax
import jax.numpy as jnp
from jax import lax
from jax.experimental import pallas as pl
from jax.experimental.pallas import tpu as pltpu
from jax.experimental.pallas import tpu_sc as plsc

B, N, M = 4, 64, 100000
QUERY_DIM = 256
BUF0 = 64
CTX_DIM = 128
HEADS, DIM_HEAD = 8, 64
INNER = HEADS * DIM_HEAD
SCALE = DIM_HEAD ** (-0.5)

NW = 32
WPB = NW // B

TC_BLK = 2048
TC_NBLK = 0
TC_ROWS = TC_BLK * TC_NBLK
SC_ROWS = M - TC_ROWS

RPW = SC_ROWS // WPB
CHUNK = 512
NFULL = RPW // CHUNK
TAIL = RPW - NFULL * CHUNK
TAIL_G = (TAIL + 15) // 16
DUN = 8


def _proj_body(x_ref, wq_ref, wqe_ref, e_ref):
    S = (lax.broadcasted_iota(jnp.int32, (B, B * N), 1)
         == lax.broadcasted_iota(jnp.int32, (B, B * N), 0) * N).astype(
             jnp.float32)
    x0 = jnp.dot(S, x_ref[...], preferred_element_type=jnp.float32)
    q0 = jnp.dot(x0, wq_ref[...], preferred_element_type=jnp.float32)
    e_ref[...] = jnp.dot(q0, wqe_ref[...], preferred_element_type=jnp.float32)


_proj = pl.pallas_call(
    _proj_body,
    out_shape=jax.ShapeDtypeStruct((B, BUF0), jnp.float32),
)


def _upd(st, x, ix):
    m1, i1, m2, i2 = st
    lt1 = x < m1
    lt2 = x < m2
    m2n = jnp.where(lt1, m1, jnp.where(lt2, x, m2))
    i2n = jnp.where(lt1, i1, jnp.where(lt2, ix, i2))
    return (jnp.where(lt1, x, m1), jnp.where(lt1, ix, i1), m2n, i2n)


_sc_mesh = plsc.VectorSubcoreMesh(core_axis_name="c", subcore_axis_name="s")


@functools.partial(
    pl.kernel,
    out_type=(
        jax.ShapeDtypeStruct((NW, 128), jnp.float32),
        jax.ShapeDtypeStruct((NW, 128), jnp.int32),
    ),
    mesh=_sc_mesh,
    compiler_params=pltpu.CompilerParams(use_tc_tiling_on_sc=False,
                                         needs_layout_passes=False),
    scratch_types=[
        pltpu.VMEM((CHUNK, BUF0), jnp.float32),
        pltpu.VMEM((CHUNK, BUF0), jnp.float32),
        pltpu.VMEM((TAIL, BUF0), jnp.float32),
        pltpu.VMEM((BUF0,), jnp.float32),
        pltpu.VMEM((128,), jnp.float32),
        pltpu.VMEM((128,), jnp.int32),
        pltpu.SemaphoreType.DMA,
        pltpu.SemaphoreType.DMA,
        pltpu.SemaphoreType.DMA,
    ],
)
def _scan_topk(ctx_hbm, e_hbm, vals_hbm, idx_hbm,
               buf0, buf1, buft, e_v, val_v, idx_v, sem0, sem1, semt):
    wid = lax.axis_index("s") * 2 + lax.axis_index("c")
    b = wid // WPB
    row0 = TC_ROWS + (wid % WPB) * RPW

    pltpu.sync_copy(e_hbm.at[b], e_v)

    pltpu.async_copy(ctx_hbm.at[b, pl.ds(row0, CHUNK), pl.ds(0, BUF0)],
                     buf0, sem0)
    pltpu.async_copy(ctx_hbm.at[b, pl.ds(row0 + CHUNK, CHUNK), pl.ds(0, BUF0)],
                     buf1, sem1)
    pltpu.async_copy(
        ctx_hbm.at[b, pl.ds(row0 + NFULL * CHUNK, TAIL), pl.ds(0, BUF0)],
        buft, semt)

    iota = jnp.arange(16, dtype=jnp.int32)
    inf16 = jnp.full((16,), jnp.inf, jnp.float32)
    zi16 = jnp.zeros((16,), jnp.int32)
    z16 = jnp.zeros((16,), jnp.float32)
    state0 = tuple((inf16, zi16, inf16, zi16) for _ in range(4))

    def compute_chunk(buf, chunk_row0, state):
        def blk_body(blk, st):
            rowvecs = [iota + (blk * 64 + g * 16) for g in range(4)]

            def d_body(dblk, accs):
                accs = list(accs)
                for k in range(DUN):
                    dval = dblk * DUN + k
                    rot = (iota + dval) & 63
                    ev = plsc.load_gather(e_v, [rot])
                    for g in range(4):
                        xg = plsc.load_gather(buf, [rowvecs[g], rot])
                        df = xg - ev
                        accs[g] = accs[g] + df * df
                return tuple(accs)

            accs = lax.fori_loop(0, BUF0 // DUN, d_body, (z16, z16, z16, z16))
            return tuple(
                _upd(st[g], accs[g], rowvecs[g] + chunk_row0)
                for g in range(4))

        return lax.fori_loop(0, CHUNK // 64, blk_body, state)

    wait_src0 = ctx_hbm.at[0, pl.ds(0, CHUNK), pl.ds(0, BUF0)]

    def pair_body(j, state):
        c0 = 2 * j
        pltpu.make_async_copy(wait_src0, buf0, sem0).wait()
        state = compute_chunk(buf0, row0 + c0 * CHUNK, state)

        @pl.when(j < NFULL // 2 - 1)
        def _():
            pltpu.async_copy(
                ctx_hbm.at[b, pl.ds(row0 + (c0 + 2) * CHUNK, CHUNK),
                           pl.ds(0, BUF0)],
                buf0, sem0)

        pltpu.make_async_copy(wait_src0, buf1, sem1).wait()
        state = compute_chunk(buf1, row0 + (c0 + 1) * CHUNK, state)

        @pl.when(j < NFULL // 2 - 1)
        def _():
            pltpu.async_copy(
                ctx_hbm.at[b, pl.ds(row0 + (c0 + 3) * CHUNK, CHUNK),
                           pl.ds(0, BUF0)],
                buf1, sem1)

        return state

    state = lax.fori_loop(0, NFULL // 2, pair_body, state0)

    pltpu.make_async_copy(
        ctx_hbm.at[0, pl.ds(0, TAIL), pl.ds(0, BUF0)], buft, semt).wait()

    def tail_body(g, st0):
        rows = jnp.minimum(iota + g * 16, TAIL - 1)

        def d_body(dblk, acc):
            for k in range(DUN):
                dval = dblk * DUN + k
                rot = (iota + dval) & 63
                ev = plsc.load_gather(e_v, [rot])
                xg = plsc.load_gather(buft, [rows, rot])
                df = xg - ev
                acc = acc + df * df
            return acc

        acc = lax.fori_loop(0, BUF0 // DUN, d_body, z16)
        nvalid = TAIL - g * 16
        x = jnp.where(iota < nvalid, acc, jnp.inf)
        ix = row0 + NFULL * CHUNK + g * 16 + iota
        return _upd(st0, x, ix)

    st0 = lax.fori_loop(0, TAIL_G, tail_body, state[0])
    state = (st0,) + state[1:]

    for g in range(4):
        val_v[pl.ds(g * 16, 16)] = state[g][0]
        val_v[pl.ds(64 + g * 16, 16)] = state[g][2]
        idx_v[pl.ds(g * 16, 16)] = state[g][1]
        idx_v[pl.ds(64 + g * 16, 16)] = state[g][3]
    pltpu.sync_copy(val_v, vals_hbm.at[wid])
    pltpu.sync_copy(idx_v, idx_hbm.at[wid])


def _unused_tc_scan_body(ctx_ref, e_ref, vals_ref, idx_ref, bufs, sems):
    bb = pl.program_id(0)
    blk = pl.program_id(1)
    step = bb * TC_NBLK + blk
    k = lax.rem(step, 2)

    def issue(bi, ii, slot):
        pltpu.make_async_copy(
            ctx_ref.at[bi, pl.ds(ii * TC_BLK, TC_BLK), :],
            bufs.at[slot], sems.at[slot]).start()

    @pl.when(step == 0)
    def _():
        issue(0, 0, 0)

    nxt = step + 1

    @pl.when(nxt < B * TC_NBLK)
    def _():
        issue(nxt // TC_NBLK, lax.rem(nxt, TC_NBLK), lax.rem(nxt, 2))

    pltpu.make_async_copy(
        ctx_ref.at[0, pl.ds(0, TC_BLK), :],
        bufs.at[k], sems.at[k]).wait()
    C = bufs[k][:, :BUF0]
    e_row = e_ref[pl.ds(bb, 1), :]
    ones = jnp.ones((BUF0, 1), jnp.float32)
    sq = jnp.dot(C * C, ones, preferred_element_type=jnp.float32)
    ce = jnp.dot(C, e_row.reshape(BUF0, 1),
                 preferred_element_type=jnp.float32)
    ee = jnp.sum(e_row * e_row)
    s = (sq - 2.0 * ce + ee).reshape(TC_BLK // 128, 128)
    ii = (lax.broadcasted_iota(jnp.int32, s.shape, 0) * 128
          + lax.broadcasted_iota(jnp.int32, s.shape, 1)
          + blk * TC_BLK).astype(jnp.float32)
    BIG = jnp.float32(3.0e38)
    m1 = jnp.min(s)
    i1 = jnp.min(jnp.where(s == m1, ii, BIG))
    s2 = jnp.where(ii == i1, BIG, s)
    m2 = jnp.min(s2)
    i2 = jnp.min(jnp.where(s2 == m2, ii, BIG))
    vals_ref[...] = jnp.stack([m1, m2]).reshape(1, 1, 2)
    idx_ref[...] = jnp.stack([i1, i2]).astype(jnp.int32).reshape(1, 1, 2)


_tc_scan = None if TC_NBLK == 0 else pl.pallas_call(
    _unused_tc_scan_body,
    grid=(B, TC_NBLK),
    in_specs=[
        pl.BlockSpec(memory_space=pltpu.MemorySpace.HBM),
        pl.BlockSpec((B, BUF0), lambda b, i: (0, 0)),
    ],
    out_specs=(
        pl.BlockSpec((1, 1, 2), lambda b, i: (b * TC_NBLK + i, 0, 0)),
        pl.BlockSpec((1, 1, 2), lambda b, i: (b * TC_NBLK + i, 0, 0)),
    ),
    out_shape=(
        jax.ShapeDtypeStruct((B * TC_NBLK, 1, 2), jnp.float32),
        jax.ShapeDtypeStruct((B * TC_NBLK, 1, 2), jnp.int32),
    ),
    scratch_shapes=[
        pltpu.VMEM((2, TC_BLK, CTX_DIM), jnp.float32),
        pltpu.SemaphoreType.DMA((2,)),
    ],
)


def _attn_body(x_ref, wq_ref, vals_ref, idx_ref, ctx_ref, wk_ref, wv_ref,
               wo_ref, bo_ref, o_ref, rows_s, sem):
    f32 = jnp.float32
    BIG = jnp.float32(3.0e38)
    vals = vals_ref[...].reshape(B, WPB * 128)
    idxf = idx_ref[...].reshape(B, WPB * 128).astype(f32)

    m1 = jnp.min(vals, axis=1, keepdims=True)
    i1 = jnp.min(jnp.where(vals == m1, idxf, BIG), axis=1, keepdims=True)
    vals2 = jnp.where(idxf == i1, BIG, vals)
    m2 = jnp.min(vals2, axis=1, keepdims=True)
    i2 = jnp.min(jnp.where(vals2 == m2, idxf, BIG), axis=1, keepdims=True)
    idx2 = jnp.concatenate([i1, i2], axis=1).astype(jnp.int32)

    for bb in range(B):
        for j in range(2):
            s = idx2[bb, j]
            pltpu.make_async_copy(
                ctx_ref.at[bb, pl.ds(s, 1), :],
                rows_s.at[bb, pl.ds(j, 1), :], sem).start()
    for _ in range(B * 2):
        pltpu.make_async_copy(
            ctx_ref.at[0, pl.ds(0, 1), :],
            rows_s.at[0, pl.ds(0, 1), :], sem).wait()

    rows = rows_s[...]
    creps = rows[:, :, :BUF0].reshape(B * 2, BUF0)
    clabels = rows[:, :, BUF0:].reshape(B * 2, BUF0)
    k = jnp.dot(clabels, wk_ref[...],
                preferred_element_type=f32).reshape(B, 2, INNER)
    v = jnp.dot(creps, wv_ref[...],
                preferred_element_type=f32).reshape(B, 2, INNER)
    q3 = jnp.dot(x_ref[...], wq_ref[...],
                 preferred_element_type=f32).reshape(B, N, INNER)

    E = (lax.broadcasted_iota(jnp.int32, (INNER, HEADS), 0) // DIM_HEAD
         == lax.broadcasted_iota(jnp.int32, (INNER, HEADS), 1)).astype(f32)

    sims = []
    for j in range(2):
        prod = (q3 * k[:, j][:, None, :]).reshape(B * N, INNER)
        sims.append(jnp.dot(prod, E, preferred_element_type=f32) * SCALE)
    mx = jnp.maximum(sims[0], sims[1])
    p0 = jnp.exp(sims[0] - mx)
    p1 = jnp.exp(sims[1] - mx)
    den = p0 + p1
    a0 = jnp.dot(p0 / den, E.T, preferred_element_type=f32).reshape(B, N, INNER)
    a1 = jnp.dot(p1 / den, E.T, preferred_element_type=f32).reshape(B, N, INNER)
    outi = a0 * v[:, 0][:, None, :] + a1 * v[:, 1][:, None, :]
    o_ref[...] = (jnp.dot(outi.reshape(B * N, INNER), wo_ref[...],
                          preferred_element_type=f32) + bo_ref[...])


_attn = pl.pallas_call(
    _attn_body,
    in_specs=[
        pl.BlockSpec(memory_space=pltpu.VMEM),
        pl.BlockSpec(memory_space=pltpu.VMEM),
        pl.BlockSpec(memory_space=pltpu.VMEM),
        pl.BlockSpec(memory_space=pltpu.VMEM),
        pl.BlockSpec(memory_space=pltpu.MemorySpace.HBM),
        pl.BlockSpec(memory_space=pltpu.VMEM),
        pl.BlockSpec(memory_space=pltpu.VMEM),
        pl.BlockSpec(memory_space=pltpu.VMEM),
        pl.BlockSpec(memory_space=pltpu.VMEM),
    ],
    out_shape=jax.ShapeDtypeStruct((B * N, QUERY_DIM), jnp.float32),
    scratch_shapes=[
        pltpu.VMEM((B, 2, CTX_DIM), jnp.float32),
        pltpu.SemaphoreType.DMA,
    ],
)


def kernel(x, context, W_q, W_k, W_v, W_qe, W_out, b_out, topk):
    del topk
    e = _proj(x.reshape(B * N, QUERY_DIM), W_q, W_qe)
    sc_vals, sc_idx = _scan_topk(context, e)
    out = _attn(x.reshape(B * N, QUERY_DIM), W_q, sc_vals, sc_idx,
                context, W_k, W_v, W_out, b_out.reshape(1, QUERY_DIM))
    return out.reshape(B, N, QUERY_DIM)
